# Initial kernel scaffold; baseline (speedup 1.0000x reference)
import jax
import jax.numpy as jnp
from jax.experimental import pallas as pl


def kernel(xyz, points, affine_alpha, affine_beta):
    raise NotImplementedError("write your pallas kernel here")


# speedup vs baseline: 8.8308x; 8.8308x over previous
import functools

import jax
import jax.numpy as jnp
from jax import lax
from jax.experimental import pallas as pl
from jax.experimental.pallas import tpu as pltpu
from jax.experimental.pallas import tpu_sc as plsc

KNN = 24
_TR = 256
_TS = 1024
_TN = 256


def _knn_body(xq_ref, xaT_ref, idx_ref, dist_ref, jref, topk_ref):
    b = pl.program_id(0)
    S = dist_ref.shape[1]
    TR = dist_ref.shape[0]

    xq = xq_ref[0]
    xaT = xaT_ref[0]

    mm = lax.dot_general(xq, xaT, (((1,), (0,)), ((), ())),
                         preferred_element_type=jnp.float32)
    sq_q = jnp.sum(xq * xq, axis=1, keepdims=True)
    sq_a = jnp.sum(xaT * xaT, axis=0, keepdims=True)
    dist = (-2.0) * mm
    dist = dist + sq_q
    dist = dist + sq_a

    dist_ref[...] = dist
    jref[...] = lax.broadcasted_iota(jnp.int32, (TR, S), 1)

    base = b * S

    def step(k, carry):
        V = dist_ref[...]
        J = jref[...]
        m = jnp.min(V, axis=1, keepdims=True)
        jc = jnp.min(jnp.where(V == m, J, S), axis=1, keepdims=True)
        dist_ref[...] = jnp.where(J == jc, jnp.inf, V)
        liota = lax.broadcasted_iota(jnp.int32, (TR, KNN), 1)
        topk_ref[...] = jnp.where(liota == k, jc, topk_ref[...])
        return carry

    lax.fori_loop(0, KNN, step, 0, unroll=False)
    idx_ref[0] = topk_ref[...] + base


def _knn_call(xp, xpT):
    B, S, _ = xp.shape
    grid = (B, S // _TR)
    return pl.pallas_call(
        _knn_body,
        grid=grid,
        in_specs=[
            pl.BlockSpec((1, _TR, 8), lambda b, t: (b, t, 0)),
            pl.BlockSpec((1, 8, S), lambda b, t: (b, 0, 0)),
        ],
        out_specs=pl.BlockSpec((1, _TR, KNN), lambda b, t: (b, t, 0)),
        out_shape=jax.ShapeDtypeStruct((B, S, KNN), jnp.int32),
        scratch_shapes=[
            pltpu.VMEM((_TR, S), jnp.float32),
            pltpu.VMEM((_TR, S), jnp.int32),
            pltpu.VMEM((_TR, KNN), jnp.int32),
        ],
        compiler_params=pltpu.CompilerParams(
            dimension_semantics=("parallel", "parallel")),
    )(xp, xpT)


def _sc_gather(points_flat, idx_flat):
    N = idx_flat.shape[0]
    C = points_flat.shape[1]
    W = 512
    mesh = plsc.VectorSubcoreMesh(core_axis_name="c", subcore_axis_name="s")
    idx2 = idx_flat.reshape(1, N)

    @functools.partial(
        pl.kernel,
        out_type=jax.ShapeDtypeStruct((N, C), jnp.float32),
        mesh=mesh,
        compiler_params=pltpu.CompilerParams(use_tc_tiling_on_sc=False),
    )
    def gk(x_hbm, i_hbm, o_hbm):
        def body(i_vmem, o_vmem):
            pltpu.sync_copy(x_hbm.at[i_vmem.at[0]], o_vmem)

        pltpu.emit_pipeline(
            body,
            grid=(N // W,),
            in_specs=[pl.BlockSpec((1, W), index_map=lambda i: (0, i))],
            out_specs=[pl.BlockSpec((W, C), index_map=lambda i: (i, 0))],
            core_axis_name=("c", "s"),
            dimension_semantics=(pltpu.PARALLEL,),
        )(i_hbm, o_hbm)

    return gk(points_flat, idx2)


def _stats_body(g_ref, p_ref, o_ref):
    g = g_ref[0]
    p = p_ref[0]
    d = g - p[:, None, :]
    o_ref[0, 0, 0, 0] = jnp.sum(d)
    o_ref[0, 0, 0, 1] = jnp.sum(d * d)


def _stats_call(g4, points):
    B, S, K, C = g4.shape
    T = S // _TS
    return pl.pallas_call(
        _stats_body,
        grid=(B, T),
        in_specs=[
            pl.BlockSpec((1, _TS, K, C), lambda b, t: (b, t, 0, 0)),
            pl.BlockSpec((1, _TS, C), lambda b, t: (b, t, 0)),
        ],
        out_specs=pl.BlockSpec((1, 1, 1, 2), lambda b, t: (b, t, 0, 0),
                               memory_space=pltpu.SMEM),
        out_shape=jax.ShapeDtypeStruct((B, T, 1, 2), jnp.float32),
        compiler_params=pltpu.CompilerParams(
            dimension_semantics=("parallel", "parallel")),
    )(g4, points)


def _norm_body(std_ref, g_ref, p_ref, a_ref, bt_ref, o_ref):
    TN, K, C = g_ref.shape[1:]
    inv = 1.0 / (std_ref[0, 0, 0] + 1e-5)
    g = g_ref[0]
    p = p_ref[0]
    a = a_ref[...]
    bt = bt_ref[...]
    d = (g - p[:, None, :]) * inv
    nrm = d * a[None] + bt[None]
    rep = jnp.broadcast_to(p[:, None, :], (TN, K, C))
    o_ref[0] = jnp.concatenate([nrm, rep], axis=-1)


def _norm_call(std, g4, points, alpha2, beta2):
    B, S, K, C = g4.shape
    return pl.pallas_call(
        _norm_body,
        grid=(B, S // _TN),
        in_specs=[
            pl.BlockSpec((1, 1, 1), lambda b, t: (b, 0, 0),
                         memory_space=pltpu.SMEM),
            pl.BlockSpec((1, _TN, K, C), lambda b, t: (b, t, 0, 0)),
            pl.BlockSpec((1, _TN, C), lambda b, t: (b, t, 0)),
            pl.BlockSpec((1, C), lambda b, t: (0, 0)),
            pl.BlockSpec((1, C), lambda b, t: (0, 0)),
        ],
        out_specs=pl.BlockSpec((1, _TN, K, 2 * C), lambda b, t: (b, t, 0, 0)),
        out_shape=jax.ShapeDtypeStruct((B, S, K, 2 * C), jnp.float32),
        compiler_params=pltpu.CompilerParams(
            dimension_semantics=("parallel", "parallel")),
    )(std, g4, points, alpha2, beta2)


def kernel(xyz, points, affine_alpha, affine_beta):
    B, S, C = points.shape
    K = KNN
    xp = jnp.concatenate(
        [xyz, jnp.zeros((B, S, 8 - xyz.shape[2]), xyz.dtype)], axis=-1)
    xpT = jnp.transpose(xp, (0, 2, 1))

    idx = _knn_call(xp, xpT)
    g = _sc_gather(points.reshape(B * S, C), idx.reshape(-1))
    g4 = g.reshape(B, S, K, C)

    stats = _stats_call(g4, points)
    s = stats.sum(axis=(1, 2))
    n = S * K * C
    var = (s[:, 1] - s[:, 0] * s[:, 0] / n) / (n - 1)
    std = jnp.sqrt(var).reshape(B, 1, 1)

    alpha2 = affine_alpha.reshape(1, C)
    beta2 = affine_beta.reshape(1, C)
    out = _norm_call(std, g4, points, alpha2, beta2)
    return (xyz, out)


# speedup vs baseline: 9.9038x; 1.1215x over previous
import functools

import jax
import jax.numpy as jnp
from jax import lax
from jax.experimental import pallas as pl
from jax.experimental.pallas import tpu as pltpu
from jax.experimental.pallas import tpu_sc as plsc

KNN = 24
_TR = 256
_TS = 1024
_TN = 256


def _knn_body(xq_ref, xaT_ref, idx_ref, dist_ref, jref, topk_ref):
    b = pl.program_id(0)
    S = dist_ref.shape[1]
    TR = dist_ref.shape[0]

    xq = xq_ref[0]
    xaT = xaT_ref[0]

    mm = lax.dot_general(xq, xaT, (((1,), (0,)), ((), ())),
                         preferred_element_type=jnp.float32)
    sq_q = jnp.sum(xq * xq, axis=1, keepdims=True)
    sq_a = jnp.sum(xaT * xaT, axis=0, keepdims=True)
    dist = (-2.0) * mm
    dist = dist + sq_q
    dist = dist + sq_a

    dist_ref[...] = dist
    jref[...] = lax.broadcasted_iota(jnp.int32, (TR, S), 1).astype(jnp.float32)

    base = b * S
    fS = jnp.float32(S)

    def step(k, jcp):
        V = dist_ref[...]
        Jf = jref[...]
        Vm = jnp.where(Jf == jcp, jnp.inf, V)
        dist_ref[...] = Vm
        m = jnp.min(Vm, axis=1, keepdims=True)
        jc = jnp.min(jnp.where(Vm == m, Jf, fS), axis=1, keepdims=True)
        liota = lax.broadcasted_iota(jnp.int32, (TR, KNN), 1)
        topk_ref[...] = jnp.where(liota == k, jc.astype(jnp.int32),
                                  topk_ref[...])
        return jc

    lax.fori_loop(0, KNN, step, jnp.full((TR, 1), -1.0, jnp.float32),
                  unroll=False)
    idx_ref[0] = topk_ref[...] + base


def _knn_call(xp, xpT):
    B, S, _ = xp.shape
    grid = (B, S // _TR)
    return pl.pallas_call(
        _knn_body,
        grid=grid,
        in_specs=[
            pl.BlockSpec((1, _TR, 8), lambda b, t: (b, t, 0)),
            pl.BlockSpec((1, 8, S), lambda b, t: (b, 0, 0)),
        ],
        out_specs=pl.BlockSpec((1, _TR, KNN), lambda b, t: (b, t, 0)),
        out_shape=jax.ShapeDtypeStruct((B, S, KNN), jnp.int32),
        scratch_shapes=[
            pltpu.VMEM((_TR, S), jnp.float32),
            pltpu.VMEM((_TR, S), jnp.float32),
            pltpu.VMEM((_TR, KNN), jnp.int32),
        ],
        compiler_params=pltpu.CompilerParams(
            dimension_semantics=("parallel", "parallel")),
    )(xp, xpT)


def _sc_gather(points_flat, idx_flat):
    N = idx_flat.shape[0]
    C = points_flat.shape[1]
    W = 512
    mesh = plsc.VectorSubcoreMesh(core_axis_name="c", subcore_axis_name="s")
    idx2 = idx_flat.reshape(1, N)

    @functools.partial(
        pl.kernel,
        out_type=jax.ShapeDtypeStruct((N, C), jnp.float32),
        mesh=mesh,
        compiler_params=pltpu.CompilerParams(use_tc_tiling_on_sc=False),
    )
    def gk(x_hbm, i_hbm, o_hbm):
        def body(i_vmem, o_vmem):
            pltpu.sync_copy(x_hbm.at[i_vmem.at[0]], o_vmem)

        pltpu.emit_pipeline(
            body,
            grid=(N // W,),
            in_specs=[pl.BlockSpec((1, W), index_map=lambda i: (0, i))],
            out_specs=[pl.BlockSpec((W, C), index_map=lambda i: (i, 0))],
            core_axis_name=("c", "s"),
            dimension_semantics=(pltpu.PARALLEL,),
        )(i_hbm, o_hbm)

    return gk(points_flat, idx2)


def _stats_body(g_ref, p_ref, o_ref):
    g = g_ref[0]
    p = p_ref[0]
    d = g - p[:, None, :]
    o_ref[0, 0, 0, 0] = jnp.sum(d)
    o_ref[0, 0, 0, 1] = jnp.sum(d * d)


def _stats_call(g4, points):
    B, S, K, C = g4.shape
    T = S // _TS
    return pl.pallas_call(
        _stats_body,
        grid=(B, T),
        in_specs=[
            pl.BlockSpec((1, _TS, K, C), lambda b, t: (b, t, 0, 0)),
            pl.BlockSpec((1, _TS, C), lambda b, t: (b, t, 0)),
        ],
        out_specs=pl.BlockSpec((1, 1, 1, 2), lambda b, t: (b, t, 0, 0),
                               memory_space=pltpu.SMEM),
        out_shape=jax.ShapeDtypeStruct((B, T, 1, 2), jnp.float32),
        compiler_params=pltpu.CompilerParams(
            dimension_semantics=("parallel", "parallel")),
    )(g4, points)


def _norm_body(std_ref, g_ref, p_ref, a_ref, bt_ref, o_ref):
    TN, K, C = g_ref.shape[1:]
    inv = 1.0 / (std_ref[0, 0, 0] + 1e-5)
    g = g_ref[0]
    p = p_ref[0]
    a = a_ref[...]
    bt = bt_ref[...]
    d = (g - p[:, None, :]) * inv
    nrm = d * a[None] + bt[None]
    rep = jnp.broadcast_to(p[:, None, :], (TN, K, C))
    o_ref[0] = jnp.concatenate([nrm, rep], axis=-1)


def _norm_call(std, g4, points, alpha2, beta2):
    B, S, K, C = g4.shape
    return pl.pallas_call(
        _norm_body,
        grid=(B, S // _TN),
        in_specs=[
            pl.BlockSpec((1, 1, 1), lambda b, t: (b, 0, 0),
                         memory_space=pltpu.SMEM),
            pl.BlockSpec((1, _TN, K, C), lambda b, t: (b, t, 0, 0)),
            pl.BlockSpec((1, _TN, C), lambda b, t: (b, t, 0)),
            pl.BlockSpec((1, C), lambda b, t: (0, 0)),
            pl.BlockSpec((1, C), lambda b, t: (0, 0)),
        ],
        out_specs=pl.BlockSpec((1, _TN, K, 2 * C), lambda b, t: (b, t, 0, 0)),
        out_shape=jax.ShapeDtypeStruct((B, S, K, 2 * C), jnp.float32),
        compiler_params=pltpu.CompilerParams(
            dimension_semantics=("parallel", "parallel")),
    )(std, g4, points, alpha2, beta2)


def kernel(xyz, points, affine_alpha, affine_beta):
    B, S, C = points.shape
    K = KNN
    xp = jnp.concatenate(
        [xyz, jnp.zeros((B, S, 8 - xyz.shape[2]), xyz.dtype)], axis=-1)
    xpT = jnp.transpose(xp, (0, 2, 1))

    idx = _knn_call(xp, xpT)
    g = _sc_gather(points.reshape(B * S, C), idx.reshape(-1))
    g4 = g.reshape(B, S, K, C)

    stats = _stats_call(g4, points)
    s = stats.sum(axis=(1, 2))
    n = S * K * C
    var = (s[:, 1] - s[:, 0] * s[:, 0] / n) / (n - 1)
    std = jnp.sqrt(var).reshape(B, 1, 1)

    alpha2 = affine_alpha.reshape(1, C)
    beta2 = affine_beta.reshape(1, C)
    out = _norm_call(std, g4, points, alpha2, beta2)
    return (xyz, out)
